# Initial kernel scaffold; baseline (speedup 1.0000x reference)
import functools

import jax
import jax.numpy as jnp
from jax import lax
from jax.experimental import pallas as pl
from jax.experimental.pallas import tpu as pltpu
from jax.experimental.pallas import tpu_sc as plsc

N = 50000
E = 1600000
P = 4
OUT = 32

NC = 2
NS = 16
NW = NC * NS

NP = 50176
SLICE = NP // NS
FSLICE = 4 * NP // NS

EPW = E // NW
WA = 10000
WC = 5000

_mesh = plsc.VectorSubcoreMesh(core_axis_name="c", subcore_axis_name="s")


@functools.partial(
    pl.kernel,
    mesh=_mesh,
    out_type=jax.ShapeDtypeStruct((NC, NP), jnp.float32),
    scratch_types=[
        pltpu.VMEM((WA,), jnp.int32),
        pltpu.VMEM((WA,), jnp.float32),
        pltpu.VMEM_SHARED((NP,), jnp.float32),
    ],
)
def _deg_kernel(col_hbm, ew_hbm, z_hbm, degp_hbm, colv, ewv, acc):
    c = lax.axis_index("c")
    s = lax.axis_index("s")
    wid = c * NS + s
    pltpu.sync_copy(z_hbm.at[pl.ds(s * SLICE, SLICE)],
                    acc.at[pl.ds(s * SLICE, SLICE)])
    plsc.subcore_barrier()
    ebase = wid * EPW

    def chunk(k, carry):
        off = ebase + k * WA
        pltpu.sync_copy(col_hbm.at[pl.ds(off, WA)], colv)
        pltpu.sync_copy(ew_hbm.at[pl.ds(off, WA)], ewv)
        pltpu.sync_copy(ewv, acc.at[colv], add=True)
        return carry

    lax.fori_loop(0, EPW // WA, chunk, 0)
    plsc.subcore_barrier()
    pltpu.sync_copy(acc.at[pl.ds(s * SLICE, SLICE)],
                    degp_hbm.at[c, pl.ds(s * SLICE, SLICE)])


@functools.partial(
    pl.kernel,
    mesh=_mesh,
    out_type=jax.ShapeDtypeStruct((NC, 4 * NP), jnp.float32),
    scratch_types=[
        pltpu.VMEM((WC,), jnp.int32),
        pltpu.VMEM((WC,), jnp.int32),
        pltpu.VMEM((WC,), jnp.float32),
        pltpu.VMEM((WC, 4), jnp.float32),
        pltpu.VMEM((4 * WC,), jnp.float32),
        pltpu.VMEM((4 * WC,), jnp.int32),
        pltpu.VMEM_SHARED((4 * NP,), jnp.float32),
        pltpu.SemaphoreType.DMA,
    ],
)
def _msg_kernel(row_hbm, col_hbm, ew_hbm, y_hbm, z_hbm, tp_hbm,
                rowv, colv, ewv, rows, msg, eidx, acc, sem):
    c = lax.axis_index("c")
    s = lax.axis_index("s")
    wid = c * NS + s
    pltpu.sync_copy(z_hbm.at[pl.ds(s * FSLICE, FSLICE)],
                    acc.at[pl.ds(s * FSLICE, FSLICE)])
    plsc.subcore_barrier()
    lane = lax.iota(jnp.int32, 16)
    lq = lax.shift_right_logical(lane, 2)
    lm = lax.bitwise_and(lane, 3)
    ebase = wid * EPW

    def chunk(k, carry):
        off = ebase + k * WC
        pltpu.sync_copy(row_hbm.at[pl.ds(off, WC)], rowv)
        pltpu.sync_copy(col_hbm.at[pl.ds(off, WC)], colv)
        pltpu.sync_copy(ew_hbm.at[pl.ds(off, WC)], ewv)
        pltpu.async_copy(y_hbm.at[rowv], rows, sem).wait()

        def body(i, carry2):
            maj = lq + i * 4
            vr = plsc.load_gather(rows, [maj, lm])
            ve = plsc.load_gather(ewv, [maj])
            vc = plsc.load_gather(colv, [maj])
            msg[pl.ds(i * 16, 16)] = vr * ve
            eidx[pl.ds(i * 16, 16)] = vc * 4 + lm
            return carry2

        lax.fori_loop(0, WC // 4, body, 0)
        pltpu.sync_copy(msg, acc.at[eidx], add=True)
        return carry

    lax.fori_loop(0, EPW // WC, chunk, 0)
    plsc.subcore_barrier()
    pltpu.sync_copy(acc.at[pl.ds(s * FSLICE, FSLICE)],
                    tp_hbm.at[c, pl.ds(s * FSLICE, FSLICE)])


def _norm_body(degp_ref, x_ref, dinv_ref, y_ref):
    d0 = degp_ref[0:NP, :]
    d1 = degp_ref[NP:2 * NP, :]
    deg = d0 + d1 + 1.0
    dinv = lax.rsqrt(jnp.maximum(deg, 1e-12))
    dinv_ref[...] = dinv
    y_ref[...] = dinv * x_ref[...]


def _norm_call(degp2, x_p):
    return pl.pallas_call(
        _norm_body,
        out_shape=[jax.ShapeDtypeStruct((NP, 1), jnp.float32),
                   jax.ShapeDtypeStruct((NP, 4), jnp.float32)],
    )(degp2, x_p)


BN = 6272
NB = NP // BN


def _gate_body(t0_ref, t1_ref, y_ref, dinv_ref, att_ref, Wz_ref, bz_ref,
               Wlz_ref, blz_ref, Wh_ref, bh_ref, Wlh_ref, blh_ref,
               Wlin_ref, blin_ref, out_ref):
    a = att_ref[...]
    e = jnp.exp(a - jnp.max(a))
    pr = e / jnp.sum(e)
    uz = jnp.dot(Wz_ref[...], Wlz_ref[0:OUT, :])
    cz = jnp.dot(bz_ref[...], Wlz_ref[0:OUT, :]) + blz_ref[...]
    uh = jnp.dot(Wh_ref[...], Wlh_ref[0:OUT, :])
    ch = jnp.dot(bh_ref[...], Wlh_ref[0:OUT, :]) + blh_ref[...]
    t = t0_ref[...] + t1_ref[...] + y_ref[...]
    s_all = dinv_ref[...] * t
    h = jnp.zeros((BN, OUT), jnp.float32)
    for p in range(P):
        sp = s_all[:, p:p + 1]
        z = jax.nn.sigmoid(sp * uz + cz)
        ht = jnp.tanh(sp * uh + ch)
        h = h + pr[0:1, p:p + 1] * (1.0 - z) * ht
    out_ref[...] = jnp.dot(jax.nn.relu(h), Wlin_ref[...]) + blin_ref[...]


def _gate_call(t0, t1, y, dinv, att2, Wz, bz2, Wlz, blz2, Wh, bh2, Wlh,
               blh2, Wlin, blin2):
    node = lambda w: pl.BlockSpec((BN, w), lambda i: (i, 0))
    full = lambda shp: pl.BlockSpec(shp, lambda i: (0, 0))
    return pl.pallas_call(
        _gate_body,
        grid=(NB,),
        in_specs=[node(4), node(4), node(4), node(1),
                  full((1, P)), full((1, OUT)), full((1, OUT)),
                  full((2 * OUT, OUT)), full((1, OUT)), full((1, OUT)),
                  full((1, OUT)), full((2 * OUT, OUT)), full((1, OUT)),
                  full((OUT, 1)), full((1, 1))],
        out_specs=node(1),
        out_shape=jax.ShapeDtypeStruct((NP, 1), jnp.float32),
    )(t0, t1, y, dinv, att2, Wz, bz2, Wlz, blz2, Wh, bh2, Wlh, blh2,
      Wlin, blin2)


def kernel(x, edge_index, edge_weight, att, Wz, bz, Wr, br, Wh, bh,
           Wlz, blz, Wlr, blr, Wlh, blh, Wlin, blin):
    row = edge_index[0]
    col = edge_index[1]
    zA = jnp.zeros((NP,), jnp.float32)
    zC = jnp.zeros((4 * NP,), jnp.float32)
    x_p = jnp.pad(x, ((0, NP - N), (0, 0)))

    degp = _deg_kernel(col, edge_weight, zA)
    dinv, y = _norm_call(degp.reshape(2 * NP, 1), x_p)
    tp = _msg_kernel(row, col, edge_weight, y, zC)
    out_p = _gate_call(
        tp[0].reshape(NP, 4), tp[1].reshape(NP, 4), y, dinv,
        att.reshape(1, P), Wz, bz.reshape(1, OUT), Wlz, blz.reshape(1, OUT),
        Wh, bh.reshape(1, OUT), Wlh, blh.reshape(1, OUT),
        Wlin, blin.reshape(1, 1))
    return out_p[:N]


# speedup vs baseline: 249.3693x; 249.3693x over previous
import functools

import jax
import jax.numpy as jnp
from jax import lax
from jax.experimental import pallas as pl
from jax.experimental.pallas import tpu as pltpu
from jax.experimental.pallas import tpu_sc as plsc

N = 50000
E = 1600000
P = 4
OUT = 32

NC = 2
NS = 16
NW = NC * NS

NP = 50176
SLICE = NP // NS
FSLICE = 4 * NP // NS

EPW = E // NW
WA = 10000
WC = 2000

_mesh = plsc.VectorSubcoreMesh(core_axis_name="c", subcore_axis_name="s")


def _vperm(v, idx):
    return lax.gather(
        v, idx[:, None],
        dimension_numbers=lax.GatherDimensionNumbers(
            offset_dims=(), collapsed_slice_dims=(0,), start_index_map=(0,)),
        slice_sizes=(1,),
        mode=lax.GatherScatterMode.PROMISE_IN_BOUNDS)


@functools.partial(
    pl.kernel,
    mesh=_mesh,
    out_type=jax.ShapeDtypeStruct((NC * NP,), jnp.float32),
    scratch_types=[
        pltpu.VMEM((WA,), jnp.int32),
        pltpu.VMEM((WA,), jnp.float32),
        pltpu.VMEM((SLICE,), jnp.float32),
        pltpu.VMEM_SHARED((NP,), jnp.float32),
    ],
)
def _deg_kernel(col_hbm, ew_hbm, degp_hbm, colv, ewv, stage, acc):
    c = lax.axis_index("c")
    s = lax.axis_index("s")
    wid = c * NS + s

    def zro(i, carry):
        stage[pl.ds(i * 16, 16)] = jnp.zeros((16,), jnp.float32)
        return carry

    lax.fori_loop(0, SLICE // 16, zro, 0)
    pltpu.sync_copy(stage, acc.at[pl.ds(s * SLICE, SLICE)])
    plsc.subcore_barrier()
    ebase = wid * EPW

    def chunk(k, carry):
        off = ebase + k * WA
        pltpu.sync_copy(col_hbm.at[pl.ds(off, WA)], colv)
        pltpu.sync_copy(ew_hbm.at[pl.ds(off, WA)], ewv)
        pltpu.sync_copy(ewv, acc.at[colv], add=True)
        return carry

    lax.fori_loop(0, EPW // WA, chunk, 0)
    plsc.subcore_barrier()
    pltpu.sync_copy(acc.at[pl.ds(s * SLICE, SLICE)], stage)
    pltpu.sync_copy(stage, degp_hbm.at[pl.ds(c * NP + s * SLICE, SLICE)])


@functools.partial(
    pl.kernel,
    mesh=_mesh,
    out_type=jax.ShapeDtypeStruct((NC * 4 * NP,), jnp.float32),
    scratch_types=[
        pltpu.VMEM((WC,), jnp.int32),
        pltpu.VMEM((WC,), jnp.int32),
        pltpu.VMEM((WC,), jnp.float32),
        pltpu.VMEM((4 * WC,), jnp.int32),
        pltpu.VMEM((4 * WC,), jnp.int32),
        pltpu.VMEM((4 * WC,), jnp.float32),
        pltpu.VMEM((4 * WC,), jnp.float32),
        pltpu.VMEM((FSLICE,), jnp.float32),
        pltpu.VMEM_SHARED((4 * NP,), jnp.float32),
        pltpu.SemaphoreType.DMA,
    ],
)
def _msg_kernel(row_hbm, col_hbm, ew_hbm, y_hbm, tp_hbm,
                rowv, colv, ewv, gidx, eidx, ewr, msg, stage, acc, sem):
    c = lax.axis_index("c")
    s = lax.axis_index("s")
    wid = c * NS + s

    def zro(i, carry):
        stage[pl.ds(i * 16, 16)] = jnp.zeros((16,), jnp.float32)
        return carry

    lax.fori_loop(0, FSLICE // 16, zro, 0)
    pltpu.sync_copy(stage, acc.at[pl.ds(s * FSLICE, FSLICE)])
    plsc.subcore_barrier()
    lane = lax.iota(jnp.int32, 16)
    lq = lax.shift_right_logical(lane, 2)
    lm = lax.bitwise_and(lane, 3)
    ebase = wid * EPW

    def chunk(k, carry):
        off = ebase + k * WC
        pltpu.sync_copy(row_hbm.at[pl.ds(off, WC)], rowv)
        pltpu.sync_copy(col_hbm.at[pl.ds(off, WC)], colv)
        pltpu.sync_copy(ew_hbm.at[pl.ds(off, WC)], ewv)

        def expand(i, carry2):
            vr = rowv[pl.ds(i * 16, 16)]
            vc = colv[pl.ds(i * 16, 16)]
            ve = ewv[pl.ds(i * 16, 16)]
            for kk in range(4):
                sel = lq + kk * 4
                o = pl.ds(i * 64 + kk * 16, 16)
                gidx[o] = _vperm(vr, sel) + lm * NP
                eidx[o] = _vperm(vc, sel) + lm * NP
                ewr[o] = _vperm(ve, sel)
            return carry2

        lax.fori_loop(0, WC // 16, expand, 0)
        pltpu.async_copy(y_hbm.at[gidx], msg, sem).wait()

        def scale(i, carry2):
            sl = pl.ds(i * 16, 16)
            msg[sl] = msg[sl] * ewr[sl]
            return carry2

        lax.fori_loop(0, WC // 4, scale, 0)
        pltpu.sync_copy(msg, acc.at[eidx], add=True)
        return carry

    lax.fori_loop(0, EPW // WC, chunk, 0)
    plsc.subcore_barrier()
    pltpu.sync_copy(acc.at[pl.ds(s * FSLICE, FSLICE)], stage)
    pltpu.sync_copy(stage, tp_hbm.at[pl.ds(c * 4 * NP + s * FSLICE, FSLICE)])


def _norm_body(degp_ref, xt_ref, dinv_ref, yt_ref):
    deg = degp_ref[0:1, :] + degp_ref[1:2, :] + 1.0
    dinv = lax.rsqrt(jnp.maximum(deg, 1e-12))
    dinv_ref[...] = dinv
    yt_ref[...] = dinv * xt_ref[...]


def _norm_call(degp, x_t):
    return pl.pallas_call(
        _norm_body,
        out_shape=[jax.ShapeDtypeStruct((1, NP), jnp.float32),
                   jax.ShapeDtypeStruct((P, NP), jnp.float32)],
    )(degp, x_t)


BL = 6272
NB = NP // BL


def _gate_body(t0_ref, t1_ref, yt_ref, dinv_ref, att_ref, WzT_ref, bzT_ref,
               WlzT_ref, blzT_ref, WhT_ref, bhT_ref, WlhT_ref, blhT_ref,
               WlinT_ref, blin_ref, out_ref):
    a = att_ref[...]
    e = jnp.exp(a - jnp.max(a))
    pr = e / jnp.sum(e)
    uz = jnp.dot(WlzT_ref[...], WzT_ref[...])
    cz = jnp.dot(WlzT_ref[...], bzT_ref[...]) + blzT_ref[...]
    uh = jnp.dot(WlhT_ref[...], WhT_ref[...])
    ch = jnp.dot(WlhT_ref[...], bhT_ref[...]) + blhT_ref[...]
    t = t0_ref[...] + t1_ref[...] + yt_ref[...]
    s_all = dinv_ref[...] * t
    h = jnp.zeros((OUT, BL), jnp.float32)
    for p in range(P):
        sp = s_all[p:p + 1, :]
        z = jax.nn.sigmoid(uz * sp + cz)
        ht = jnp.tanh(uh * sp + ch)
        h = h + pr[0:1, p:p + 1] * (1.0 - z) * ht
    out_ref[...] = jnp.dot(WlinT_ref[...], jax.nn.relu(h)) + blin_ref[...]


def _gate_call(t0, t1, y_t, dinv, att2, WzT, bzT, WlzT, blzT, WhT, bhT,
               WlhT, blhT, WlinT, blin2):
    node = lambda r: pl.BlockSpec((r, BL), lambda i: (0, i))
    full = lambda shp: pl.BlockSpec(shp, lambda i: (0, 0))
    return pl.pallas_call(
        _gate_body,
        grid=(NB,),
        in_specs=[node(P), node(P), node(P), node(1),
                  full((1, P)), full((OUT, 1)), full((OUT, 1)),
                  full((OUT, OUT)), full((OUT, 1)), full((OUT, 1)),
                  full((OUT, 1)), full((OUT, OUT)), full((OUT, 1)),
                  full((1, OUT)), full((1, 1))],
        out_specs=node(1),
        out_shape=jax.ShapeDtypeStruct((1, NP), jnp.float32),
    )(t0, t1, y_t, dinv, att2, WzT, bzT, WlzT, blzT, WhT, bhT, WlhT, blhT,
      WlinT, blin2)


def kernel(x, edge_index, edge_weight, att, Wz, bz, Wr, br, Wh, bh,
           Wlz, blz, Wlr, blr, Wlh, blh, Wlin, blin):
    row = edge_index[0]
    col = edge_index[1]
    x_t = jnp.pad(x.T, ((0, 0), (0, NP - N)))

    degp = _deg_kernel(col, edge_weight)
    dinv, y_t = _norm_call(degp.reshape(2, NP), x_t)
    tp = _msg_kernel(row, col, edge_weight, y_t.reshape(4 * NP))
    out_t = _gate_call(
        tp[:4 * NP].reshape(P, NP), tp[4 * NP:].reshape(P, NP), y_t, dinv,
        att.reshape(1, P),
        Wz.reshape(OUT, 1), bz.reshape(OUT, 1),
        Wlz[:OUT].T, blz.reshape(OUT, 1),
        Wh.reshape(OUT, 1), bh.reshape(OUT, 1),
        Wlh[:OUT].T, blh.reshape(OUT, 1),
        Wlin.reshape(1, OUT), blin.reshape(1, 1))
    return out_t.reshape(NP, 1)[:N]


# speedup vs baseline: 343.5609x; 1.3777x over previous
import functools

import jax
import jax.numpy as jnp
from jax import lax
from jax.experimental import pallas as pl
from jax.experimental.pallas import tpu as pltpu
from jax.experimental.pallas import tpu_sc as plsc

N = 50000
E = 1600000
P = 4
OUT = 32

NC = 2
NS = 16
NW = NC * NS

NP = 50176
SLICE = NP // NS
FSLICE = 4 * NP // NS

EPW = E // NW
WA = 10000
WC = 2000

_mesh = plsc.VectorSubcoreMesh(core_axis_name="c", subcore_axis_name="s")


def _vperm(v, idx):
    return lax.gather(
        v, idx[:, None],
        dimension_numbers=lax.GatherDimensionNumbers(
            offset_dims=(), collapsed_slice_dims=(0,), start_index_map=(0,)),
        slice_sizes=(1,),
        mode=lax.GatherScatterMode.PROMISE_IN_BOUNDS)


@functools.partial(
    pl.kernel,
    mesh=_mesh,
    out_type=jax.ShapeDtypeStruct((NC * NP,), jnp.float32),
    scratch_types=[
        pltpu.VMEM((WA,), jnp.int32),
        pltpu.VMEM((WA,), jnp.float32),
        pltpu.VMEM((SLICE,), jnp.float32),
        pltpu.VMEM_SHARED((NP,), jnp.float32),
    ],
)
def _deg_kernel(col_hbm, ew_hbm, degp_hbm, colv, ewv, stage, acc):
    c = lax.axis_index("c")
    s = lax.axis_index("s")
    wid = c * NS + s

    def zro(i, carry):
        stage[pl.ds(i * 16, 16)] = jnp.zeros((16,), jnp.float32)
        return carry

    lax.fori_loop(0, SLICE // 16, zro, 0)
    pltpu.sync_copy(stage, acc.at[pl.ds(s * SLICE, SLICE)])
    plsc.subcore_barrier()
    ebase = wid * EPW

    def chunk(k, carry):
        off = ebase + k * WA
        pltpu.sync_copy(col_hbm.at[pl.ds(off, WA)], colv)
        pltpu.sync_copy(ew_hbm.at[pl.ds(off, WA)], ewv)
        pltpu.sync_copy(ewv, acc.at[colv], add=True)
        return carry

    lax.fori_loop(0, EPW // WA, chunk, 0)
    plsc.subcore_barrier()
    pltpu.sync_copy(acc.at[pl.ds(s * SLICE, SLICE)], stage)
    pltpu.sync_copy(stage, degp_hbm.at[pl.ds(c * NP + s * SLICE, SLICE)])


NCH = EPW // WC


@functools.partial(
    pl.kernel,
    mesh=_mesh,
    out_type=jax.ShapeDtypeStruct((NC * 4 * NP,), jnp.float32),
    scratch_types=[
        pltpu.VMEM((WC,), jnp.int32),
        pltpu.VMEM((WC,), jnp.int32),
        pltpu.VMEM((WC,), jnp.int32),
        pltpu.VMEM((WC,), jnp.int32),
        pltpu.VMEM((WC,), jnp.float32),
        pltpu.VMEM((WC,), jnp.float32),
        pltpu.VMEM((4 * WC,), jnp.int32),
        pltpu.VMEM((4 * WC,), jnp.int32),
        pltpu.VMEM((4 * WC,), jnp.int32),
        pltpu.VMEM((4 * WC,), jnp.int32),
        pltpu.VMEM((4 * WC,), jnp.float32),
        pltpu.VMEM((4 * WC,), jnp.float32),
        pltpu.VMEM((4 * WC,), jnp.float32),
        pltpu.VMEM((4 * WC,), jnp.float32),
        pltpu.VMEM((FSLICE,), jnp.float32),
        pltpu.VMEM_SHARED((4 * NP,), jnp.float32),
        pltpu.SemaphoreType.DMA,
        pltpu.SemaphoreType.DMA,
        pltpu.SemaphoreType.DMA,
        pltpu.SemaphoreType.DMA,
    ],
)
def _msg_kernel(row_hbm, col_hbm, ew_hbm, y_hbm, tp_hbm,
                rowv0, rowv1, colv0, colv1, ewv0, ewv1,
                gidx0, gidx1, eidx0, eidx1, ewr0, ewr1, msg0, msg1,
                stage, acc, gsem0, gsem1, ssem0, ssem1):
    rowv, colv, ewv = [rowv0, rowv1], [colv0, colv1], [ewv0, ewv1]
    gidx, eidx = [gidx0, gidx1], [eidx0, eidx1]
    ewr, msg = [ewr0, ewr1], [msg0, msg1]
    gsem, ssem = [gsem0, gsem1], [ssem0, ssem1]
    c = lax.axis_index("c")
    s = lax.axis_index("s")
    wid = c * NS + s

    def zro(i, carry):
        stage[pl.ds(i * 16, 16)] = jnp.zeros((16,), jnp.float32)
        return carry

    lax.fori_loop(0, FSLICE // 16, zro, 0)
    pltpu.sync_copy(stage, acc.at[pl.ds(s * FSLICE, FSLICE)])
    plsc.subcore_barrier()
    lane = lax.iota(jnp.int32, 16)
    lq = lax.shift_right_logical(lane, 2)
    lm = lax.bitwise_and(lane, 3)
    ebase = wid * EPW

    def load(k):
        b, off = k % 2, ebase + k * WC
        pltpu.sync_copy(row_hbm.at[pl.ds(off, WC)], rowv[b])
        pltpu.sync_copy(col_hbm.at[pl.ds(off, WC)], colv[b])
        pltpu.sync_copy(ew_hbm.at[pl.ds(off, WC)], ewv[b])

    def expand(k):
        b = k % 2

        def body(i, carry):
            vr = rowv[b][pl.ds(i * 16, 16)]
            vc = colv[b][pl.ds(i * 16, 16)]
            ve = ewv[b][pl.ds(i * 16, 16)]
            for kk in range(4):
                sel = lq + kk * 4
                o = pl.ds(i * 64 + kk * 16, 16)
                gidx[b][o] = _vperm(vr, sel) + lm * NP
                eidx[b][o] = _vperm(vc, sel) + lm * NP
                ewr[b][o] = _vperm(ve, sel)
            return carry

        lax.fori_loop(0, WC // 16, body, 0)

    def scale(k):
        b = k % 2

        def body(i, carry):
            sl = pl.ds(i * 16, 16)
            msg[b][sl] = msg[b][sl] * ewr[b][sl]
            return carry

        lax.fori_loop(0, WC // 4, body, 0)

    def start_gather(k):
        b = k % 2
        return pltpu.async_copy(y_hbm.at[gidx[b]], msg[b], gsem[b])

    load(0)
    expand(0)
    gd = start_gather(0)
    sd = None
    for k in range(NCH):
        b = k % 2
        if sd is not None:
            sd.wait()
        if k + 1 < NCH:
            load(k + 1)
            expand(k + 1)
            gd_next = start_gather(k + 1)
        gd.wait()
        scale(k)
        sd = pltpu.async_copy(msg[b], acc.at[eidx[b]], ssem[b], add=True)
        if k + 1 < NCH:
            gd = gd_next
    sd.wait()
    plsc.subcore_barrier()
    pltpu.sync_copy(acc.at[pl.ds(s * FSLICE, FSLICE)], stage)
    pltpu.sync_copy(stage, tp_hbm.at[pl.ds(c * 4 * NP + s * FSLICE, FSLICE)])


def _norm_body(degp_ref, xt_ref, dinv_ref, yt_ref):
    deg = degp_ref[0:1, :] + degp_ref[1:2, :] + 1.0
    dinv = lax.rsqrt(jnp.maximum(deg, 1e-12))
    dinv_ref[...] = dinv
    yt_ref[...] = dinv * xt_ref[...]


def _norm_call(degp, x_t):
    return pl.pallas_call(
        _norm_body,
        out_shape=[jax.ShapeDtypeStruct((1, NP), jnp.float32),
                   jax.ShapeDtypeStruct((P, NP), jnp.float32)],
    )(degp, x_t)


BL = 6272
NB = NP // BL


def _gate_body(t0_ref, t1_ref, yt_ref, dinv_ref, att_ref, WzT_ref, bzT_ref,
               WlzT_ref, blzT_ref, WhT_ref, bhT_ref, WlhT_ref, blhT_ref,
               WlinT_ref, blin_ref, out_ref):
    a = att_ref[...]
    e = jnp.exp(a - jnp.max(a))
    pr = e / jnp.sum(e)
    uz = jnp.dot(WlzT_ref[...], WzT_ref[...])
    cz = jnp.dot(WlzT_ref[...], bzT_ref[...]) + blzT_ref[...]
    uh = jnp.dot(WlhT_ref[...], WhT_ref[...])
    ch = jnp.dot(WlhT_ref[...], bhT_ref[...]) + blhT_ref[...]
    t = t0_ref[...] + t1_ref[...] + yt_ref[...]
    s_all = dinv_ref[...] * t
    h = jnp.zeros((OUT, BL), jnp.float32)
    for p in range(P):
        sp = s_all[p:p + 1, :]
        z = jax.nn.sigmoid(uz * sp + cz)
        ht = jnp.tanh(uh * sp + ch)
        h = h + pr[0:1, p:p + 1] * (1.0 - z) * ht
    out_ref[...] = jnp.dot(WlinT_ref[...], jax.nn.relu(h)) + blin_ref[...]


def _gate_call(t0, t1, y_t, dinv, att2, WzT, bzT, WlzT, blzT, WhT, bhT,
               WlhT, blhT, WlinT, blin2):
    node = lambda r: pl.BlockSpec((r, BL), lambda i: (0, i))
    full = lambda shp: pl.BlockSpec(shp, lambda i: (0, 0))
    return pl.pallas_call(
        _gate_body,
        grid=(NB,),
        in_specs=[node(P), node(P), node(P), node(1),
                  full((1, P)), full((OUT, 1)), full((OUT, 1)),
                  full((OUT, OUT)), full((OUT, 1)), full((OUT, 1)),
                  full((OUT, 1)), full((OUT, OUT)), full((OUT, 1)),
                  full((1, OUT)), full((1, 1))],
        out_specs=node(1),
        out_shape=jax.ShapeDtypeStruct((1, NP), jnp.float32),
    )(t0, t1, y_t, dinv, att2, WzT, bzT, WlzT, blzT, WhT, bhT, WlhT, blhT,
      WlinT, blin2)


def kernel(x, edge_index, edge_weight, att, Wz, bz, Wr, br, Wh, bh,
           Wlz, blz, Wlr, blr, Wlh, blh, Wlin, blin):
    row = edge_index[0]
    col = edge_index[1]
    x_t = jnp.pad(x.T, ((0, 0), (0, NP - N)))

    degp = _deg_kernel(col, edge_weight)
    dinv, y_t = _norm_call(degp.reshape(2, NP), x_t)
    tp = _msg_kernel(row, col, edge_weight, y_t.reshape(4 * NP))
    out_t = _gate_call(
        tp[:4 * NP].reshape(P, NP), tp[4 * NP:].reshape(P, NP), y_t, dinv,
        att.reshape(1, P),
        Wz.reshape(OUT, 1), bz.reshape(OUT, 1),
        Wlz[:OUT].T, blz.reshape(OUT, 1),
        Wh.reshape(OUT, 1), bh.reshape(OUT, 1),
        Wlh[:OUT].T, blh.reshape(OUT, 1),
        Wlin.reshape(1, OUT), blin.reshape(1, 1))
    return out_t.reshape(NP, 1)[:N]


# speedup vs baseline: 406.9605x; 1.1845x over previous
import functools

import jax
import jax.numpy as jnp
from jax import lax
from jax.experimental import pallas as pl
from jax.experimental.pallas import tpu as pltpu
from jax.experimental.pallas import tpu_sc as plsc

N = 50000
E = 1600000
P = 4
OUT = 32

NC = 2
NS = 16
NW = NC * NS

NP = 50176
SLICE = NP // NS
FSLICE = 4 * NP // NS

EPW = E // NW
WA = 10000
WC = 2000

_mesh = plsc.VectorSubcoreMesh(core_axis_name="c", subcore_axis_name="s")


def _vperm(v, idx):
    return lax.gather(
        v, idx[:, None],
        dimension_numbers=lax.GatherDimensionNumbers(
            offset_dims=(), collapsed_slice_dims=(0,), start_index_map=(0,)),
        slice_sizes=(1,),
        mode=lax.GatherScatterMode.PROMISE_IN_BOUNDS)


@functools.partial(
    pl.kernel,
    mesh=_mesh,
    out_type=jax.ShapeDtypeStruct((NC * NP,), jnp.float32),
    scratch_types=[
        pltpu.VMEM((WA,), jnp.int32),
        pltpu.VMEM((WA,), jnp.float32),
        pltpu.VMEM((SLICE,), jnp.float32),
        pltpu.VMEM_SHARED((NP,), jnp.float32),
    ],
)
def _deg_kernel(col_hbm, ew_hbm, degp_hbm, colv, ewv, stage, acc):
    c = lax.axis_index("c")
    s = lax.axis_index("s")
    wid = c * NS + s

    def zro(i, carry):
        stage[pl.ds(i * 16, 16)] = jnp.zeros((16,), jnp.float32)
        return carry

    lax.fori_loop(0, SLICE // 16, zro, 0)
    pltpu.sync_copy(stage, acc.at[pl.ds(s * SLICE, SLICE)])
    plsc.subcore_barrier()
    ebase = wid * EPW

    def chunk(k, carry):
        off = ebase + k * WA
        pltpu.sync_copy(col_hbm.at[pl.ds(off, WA)], colv)
        pltpu.sync_copy(ew_hbm.at[pl.ds(off, WA)], ewv)
        pltpu.sync_copy(ewv, acc.at[colv], add=True)
        return carry

    lax.fori_loop(0, EPW // WA, chunk, 0)
    plsc.subcore_barrier()
    pltpu.sync_copy(acc.at[pl.ds(s * SLICE, SLICE)], stage)
    pltpu.sync_copy(stage, degp_hbm.at[pl.ds(c * NP + s * SLICE, SLICE)])


NCH = EPW // WC


@functools.partial(
    pl.kernel,
    mesh=_mesh,
    out_type=jax.ShapeDtypeStruct((NC * 4 * NP,), jnp.float32),
    scratch_types=[
        pltpu.VMEM((WC,), jnp.int32),
        pltpu.VMEM((WC,), jnp.int32),
        pltpu.VMEM((WC,), jnp.int32),
        pltpu.VMEM((WC,), jnp.int32),
        pltpu.VMEM((WC,), jnp.float32),
        pltpu.VMEM((WC,), jnp.float32),
        pltpu.VMEM((4 * WC,), jnp.int32),
        pltpu.VMEM((4 * WC,), jnp.int32),
        pltpu.VMEM((4 * WC,), jnp.int32),
        pltpu.VMEM((4 * WC,), jnp.int32),
        pltpu.VMEM((4 * WC,), jnp.float32),
        pltpu.VMEM((4 * WC,), jnp.float32),
        pltpu.VMEM((4 * WC,), jnp.float32),
        pltpu.VMEM((4 * WC,), jnp.float32),
        pltpu.VMEM((FSLICE,), jnp.float32),
        pltpu.VMEM_SHARED((4 * NP,), jnp.float32),
        pltpu.VMEM_SHARED((4 * NP,), jnp.float32),
        pltpu.SemaphoreType.DMA,
        pltpu.SemaphoreType.DMA,
        pltpu.SemaphoreType.DMA,
        pltpu.SemaphoreType.DMA,
    ],
)
def _msg_kernel(row_hbm, col_hbm, ew_hbm, y_hbm, tp_hbm,
                rowv0, rowv1, colv0, colv1, ewv0, ewv1,
                gidx0, gidx1, eidx0, eidx1, ewr0, ewr1, msg0, msg1,
                stage, acc, ysh, gsem0, gsem1, ssem0, ssem1):
    rowv, colv, ewv = [rowv0, rowv1], [colv0, colv1], [ewv0, ewv1]
    gidx, eidx = [gidx0, gidx1], [eidx0, eidx1]
    ewr, msg = [ewr0, ewr1], [msg0, msg1]
    gsem, ssem = [gsem0, gsem1], [ssem0, ssem1]
    c = lax.axis_index("c")
    s = lax.axis_index("s")
    wid = c * NS + s

    pltpu.sync_copy(y_hbm.at[pl.ds(s * FSLICE, FSLICE)], stage)
    pltpu.sync_copy(stage, ysh.at[pl.ds(s * FSLICE, FSLICE)])

    def zro(i, carry):
        stage[pl.ds(i * 16, 16)] = jnp.zeros((16,), jnp.float32)
        return carry

    lax.fori_loop(0, FSLICE // 16, zro, 0)
    pltpu.sync_copy(stage, acc.at[pl.ds(s * FSLICE, FSLICE)])
    plsc.subcore_barrier()
    lane = lax.iota(jnp.int32, 16)
    lq = lax.shift_right_logical(lane, 2)
    lm = lax.bitwise_and(lane, 3)
    ebase = wid * EPW

    def load(k):
        b, off = k % 2, ebase + k * WC
        pltpu.sync_copy(row_hbm.at[pl.ds(off, WC)], rowv[b])
        pltpu.sync_copy(col_hbm.at[pl.ds(off, WC)], colv[b])
        pltpu.sync_copy(ew_hbm.at[pl.ds(off, WC)], ewv[b])

    def expand(k):
        b = k % 2

        def body(i, carry):
            vr = rowv[b][pl.ds(i * 16, 16)]
            vc = colv[b][pl.ds(i * 16, 16)]
            ve = ewv[b][pl.ds(i * 16, 16)]
            for kk in range(4):
                sel = lq + kk * 4
                o = pl.ds(i * 64 + kk * 16, 16)
                gidx[b][o] = _vperm(vr, sel) + lm * NP
                eidx[b][o] = _vperm(vc, sel) + lm * NP
                ewr[b][o] = _vperm(ve, sel)
            return carry

        lax.fori_loop(0, WC // 16, body, 0)

    def scale(k):
        b = k % 2

        def body(i, carry):
            sl = pl.ds(i * 16, 16)
            msg[b][sl] = msg[b][sl] * ewr[b][sl]
            return carry

        lax.fori_loop(0, WC // 4, body, 0)

    def start_gather(k):
        b = k % 2
        return pltpu.async_copy(ysh.at[gidx[b]], msg[b], gsem[b])

    load(0)
    expand(0)
    gd = start_gather(0)
    sd = None
    for k in range(NCH):
        b = k % 2
        if sd is not None:
            sd.wait()
        if k + 1 < NCH:
            load(k + 1)
            expand(k + 1)
            gd_next = start_gather(k + 1)
        gd.wait()
        scale(k)
        sd = pltpu.async_copy(msg[b], acc.at[eidx[b]], ssem[b], add=True)
        if k + 1 < NCH:
            gd = gd_next
    sd.wait()
    plsc.subcore_barrier()
    pltpu.sync_copy(acc.at[pl.ds(s * FSLICE, FSLICE)], stage)
    pltpu.sync_copy(stage, tp_hbm.at[pl.ds(c * 4 * NP + s * FSLICE, FSLICE)])


def _norm_body(degp_ref, xt_ref, dinv_ref, yt_ref):
    deg = degp_ref[0:1, :] + degp_ref[1:2, :] + 1.0
    dinv = lax.rsqrt(jnp.maximum(deg, 1e-12))
    dinv_ref[...] = dinv
    yt_ref[...] = dinv * xt_ref[...]


def _norm_call(degp, x_t):
    return pl.pallas_call(
        _norm_body,
        out_shape=[jax.ShapeDtypeStruct((1, NP), jnp.float32),
                   jax.ShapeDtypeStruct((P, NP), jnp.float32)],
    )(degp, x_t)


BL = 6272
NB = NP // BL


def _gate_body(t0_ref, t1_ref, yt_ref, dinv_ref, att_ref, WzT_ref, bzT_ref,
               WlzT_ref, blzT_ref, WhT_ref, bhT_ref, WlhT_ref, blhT_ref,
               WlinT_ref, blin_ref, out_ref):
    a = att_ref[...]
    e = jnp.exp(a - jnp.max(a))
    pr = e / jnp.sum(e)
    uz = jnp.dot(WlzT_ref[...], WzT_ref[...])
    cz = jnp.dot(WlzT_ref[...], bzT_ref[...]) + blzT_ref[...]
    uh = jnp.dot(WlhT_ref[...], WhT_ref[...])
    ch = jnp.dot(WlhT_ref[...], bhT_ref[...]) + blhT_ref[...]
    t = t0_ref[...] + t1_ref[...] + yt_ref[...]
    s_all = dinv_ref[...] * t
    h = jnp.zeros((OUT, BL), jnp.float32)
    for p in range(P):
        sp = s_all[p:p + 1, :]
        z = jax.nn.sigmoid(uz * sp + cz)
        ht = jnp.tanh(uh * sp + ch)
        h = h + pr[0:1, p:p + 1] * (1.0 - z) * ht
    out_ref[...] = jnp.dot(WlinT_ref[...], jax.nn.relu(h)) + blin_ref[...]


def _gate_call(t0, t1, y_t, dinv, att2, WzT, bzT, WlzT, blzT, WhT, bhT,
               WlhT, blhT, WlinT, blin2):
    node = lambda r: pl.BlockSpec((r, BL), lambda i: (0, i))
    full = lambda shp: pl.BlockSpec(shp, lambda i: (0, 0))
    return pl.pallas_call(
        _gate_body,
        grid=(NB,),
        in_specs=[node(P), node(P), node(P), node(1),
                  full((1, P)), full((OUT, 1)), full((OUT, 1)),
                  full((OUT, OUT)), full((OUT, 1)), full((OUT, 1)),
                  full((OUT, 1)), full((OUT, OUT)), full((OUT, 1)),
                  full((1, OUT)), full((1, 1))],
        out_specs=node(1),
        out_shape=jax.ShapeDtypeStruct((1, NP), jnp.float32),
    )(t0, t1, y_t, dinv, att2, WzT, bzT, WlzT, blzT, WhT, bhT, WlhT, blhT,
      WlinT, blin2)


def kernel(x, edge_index, edge_weight, att, Wz, bz, Wr, br, Wh, bh,
           Wlz, blz, Wlr, blr, Wlh, blh, Wlin, blin):
    row = edge_index[0]
    col = edge_index[1]
    x_t = jnp.pad(x.T, ((0, 0), (0, NP - N)))

    degp = _deg_kernel(col, edge_weight)
    dinv, y_t = _norm_call(degp.reshape(2, NP), x_t)
    tp = _msg_kernel(row, col, edge_weight, y_t.reshape(4 * NP))
    out_t = _gate_call(
        tp[:4 * NP].reshape(P, NP), tp[4 * NP:].reshape(P, NP), y_t, dinv,
        att.reshape(1, P),
        Wz.reshape(OUT, 1), bz.reshape(OUT, 1),
        Wlz[:OUT].T, blz.reshape(OUT, 1),
        Wh.reshape(OUT, 1), bh.reshape(OUT, 1),
        Wlh[:OUT].T, blh.reshape(OUT, 1),
        Wlin.reshape(1, OUT), blin.reshape(1, 1))
    return out_t.reshape(NP, 1)[:N]


# speedup vs baseline: 433.2946x; 1.0647x over previous
import functools

import jax
import jax.numpy as jnp
from jax import lax
from jax.experimental import pallas as pl
from jax.experimental.pallas import tpu as pltpu
from jax.experimental.pallas import tpu_sc as plsc

N = 50000
E = 1600000
P = 4
OUT = 32

NC = 2
NS = 16
NW = NC * NS

NP = 50176
SLICE = NP // NS
FSLICE = 4 * NP // NS

EPW = E // NW
WA = 10000
WC = 2000

_mesh = plsc.VectorSubcoreMesh(core_axis_name="c", subcore_axis_name="s")


def _vperm(v, idx):
    return lax.gather(
        v, idx[:, None],
        dimension_numbers=lax.GatherDimensionNumbers(
            offset_dims=(), collapsed_slice_dims=(0,), start_index_map=(0,)),
        slice_sizes=(1,),
        mode=lax.GatherScatterMode.PROMISE_IN_BOUNDS)


@functools.partial(
    pl.kernel,
    mesh=_mesh,
    out_type=jax.ShapeDtypeStruct((NC * NP,), jnp.float32),
    scratch_types=[
        pltpu.VMEM((WA,), jnp.int32),
        pltpu.VMEM((WA,), jnp.float32),
        pltpu.VMEM((SLICE,), jnp.float32),
        pltpu.VMEM_SHARED((NP,), jnp.float32),
    ],
)
def _deg_kernel(col_hbm, ew_hbm, degp_hbm, colv, ewv, stage, acc):
    c = lax.axis_index("c")
    s = lax.axis_index("s")
    wid = c * NS + s

    def zro(i, carry):
        stage[pl.ds(i * 16, 16)] = jnp.zeros((16,), jnp.float32)
        return carry

    lax.fori_loop(0, SLICE // 16, zro, 0)
    pltpu.sync_copy(stage, acc.at[pl.ds(s * SLICE, SLICE)])
    plsc.subcore_barrier()
    ebase = wid * EPW

    def chunk(k, carry):
        off = ebase + k * WA
        pltpu.sync_copy(col_hbm.at[pl.ds(off, WA)], colv)
        pltpu.sync_copy(ew_hbm.at[pl.ds(off, WA)], ewv)
        pltpu.sync_copy(ewv, acc.at[colv], add=True)
        return carry

    lax.fori_loop(0, EPW // WA, chunk, 0)
    plsc.subcore_barrier()
    pltpu.sync_copy(acc.at[pl.ds(s * SLICE, SLICE)], stage)
    pltpu.sync_copy(stage, degp_hbm.at[pl.ds(c * NP + s * SLICE, SLICE)])


NCH = EPW // WC


@functools.partial(
    pl.kernel,
    mesh=_mesh,
    out_type=jax.ShapeDtypeStruct((NC * 4 * NP,), jnp.float32),
    scratch_types=[
        pltpu.VMEM((WC,), jnp.int32),
        pltpu.VMEM((WC,), jnp.int32),
        pltpu.VMEM((WC,), jnp.int32),
        pltpu.VMEM((WC,), jnp.int32),
        pltpu.VMEM((WC,), jnp.float32),
        pltpu.VMEM((WC,), jnp.float32),
        pltpu.VMEM((4 * WC,), jnp.int32),
        pltpu.VMEM((4 * WC,), jnp.int32),
        pltpu.VMEM((4 * WC,), jnp.int32),
        pltpu.VMEM((4 * WC,), jnp.int32),
        pltpu.VMEM((4 * WC,), jnp.float32),
        pltpu.VMEM((4 * WC,), jnp.float32),
        pltpu.VMEM((4 * WC,), jnp.float32),
        pltpu.VMEM((4 * WC,), jnp.float32),
        pltpu.VMEM((FSLICE,), jnp.float32),
        pltpu.VMEM_SHARED((4 * NP,), jnp.float32),
        pltpu.VMEM_SHARED((4 * NP,), jnp.float32),
        pltpu.SemaphoreType.DMA,
        pltpu.SemaphoreType.DMA,
        pltpu.SemaphoreType.DMA,
        pltpu.SemaphoreType.DMA,
    ],
)
def _msg_kernel(row_hbm, col_hbm, ew_hbm, y_hbm, tp_hbm,
                rowv0, rowv1, colv0, colv1, ewv0, ewv1,
                gidx0, gidx1, eidx0, eidx1, ewr0, ewr1, msg0, msg1,
                stage, acc, ysh, gsem0, gsem1, ssem0, ssem1):
    rowv, colv, ewv = [rowv0, rowv1], [colv0, colv1], [ewv0, ewv1]
    gidx, eidx = [gidx0, gidx1], [eidx0, eidx1]
    ewr, msg = [ewr0, ewr1], [msg0, msg1]
    gsem, ssem = [gsem0, gsem1], [ssem0, ssem1]
    c = lax.axis_index("c")
    s = lax.axis_index("s")
    wid = c * NS + s

    pltpu.sync_copy(y_hbm.at[pl.ds(s * FSLICE, FSLICE)], stage)
    pltpu.sync_copy(stage, ysh.at[pl.ds(s * FSLICE, FSLICE)])

    def zro(i, carry):
        stage[pl.ds(i * 16, 16)] = jnp.zeros((16,), jnp.float32)
        return carry

    lax.fori_loop(0, FSLICE // 16, zro, 0)
    pltpu.sync_copy(stage, acc.at[pl.ds(s * FSLICE, FSLICE)])
    plsc.subcore_barrier()
    lane = lax.iota(jnp.int32, 16)
    lq = lax.shift_right_logical(lane, 2)
    lm = lax.bitwise_and(lane, 3)
    ebase = wid * EPW

    def load(k):
        b, off = k % 2, ebase + k * WC
        pltpu.sync_copy(row_hbm.at[pl.ds(off, WC)], rowv[b])
        pltpu.sync_copy(col_hbm.at[pl.ds(off, WC)], colv[b])
        pltpu.sync_copy(ew_hbm.at[pl.ds(off, WC)], ewv[b])

    def expand(k):
        b = k % 2

        def body(i, carry):
            vr = rowv[b][pl.ds(i * 16, 16)]
            vc = colv[b][pl.ds(i * 16, 16)]
            ve = ewv[b][pl.ds(i * 16, 16)]
            for p in range(4):
                o = pl.ds(p * WC + i * 16, 16)
                gidx[b][o] = vr + p * NP
                eidx[b][o] = vc + p * NP
                ewr[b][o] = ve
            return carry

        lax.fori_loop(0, WC // 16, body, 0)

    def scale(k):
        b = k % 2

        def body(i, carry):
            sl = pl.ds(i * 16, 16)
            msg[b][sl] = msg[b][sl] * ewr[b][sl]
            return carry

        lax.fori_loop(0, WC // 4, body, 0)

    def start_gather(k):
        b = k % 2
        return pltpu.async_copy(ysh.at[gidx[b]], msg[b], gsem[b])

    load(0)
    expand(0)
    gd = start_gather(0)
    sd = None
    for k in range(NCH):
        b = k % 2
        if sd is not None:
            sd.wait()
        if k + 1 < NCH:
            load(k + 1)
            expand(k + 1)
            gd_next = start_gather(k + 1)
        gd.wait()
        scale(k)
        sd = pltpu.async_copy(msg[b], acc.at[eidx[b]], ssem[b], add=True)
        if k + 1 < NCH:
            gd = gd_next
    sd.wait()
    plsc.subcore_barrier()
    pltpu.sync_copy(acc.at[pl.ds(s * FSLICE, FSLICE)], stage)
    pltpu.sync_copy(stage, tp_hbm.at[pl.ds(c * 4 * NP + s * FSLICE, FSLICE)])


def _norm_body(degp_ref, xt_ref, dinv_ref, yt_ref):
    deg = degp_ref[0:1, :] + degp_ref[1:2, :] + 1.0
    dinv = lax.rsqrt(jnp.maximum(deg, 1e-12))
    dinv_ref[...] = dinv
    yt_ref[...] = dinv * xt_ref[...]


def _norm_call(degp, x_t):
    return pl.pallas_call(
        _norm_body,
        out_shape=[jax.ShapeDtypeStruct((1, NP), jnp.float32),
                   jax.ShapeDtypeStruct((P, NP), jnp.float32)],
    )(degp, x_t)


BL = 6272
NB = NP // BL


def _gate_body(t0_ref, t1_ref, yt_ref, dinv_ref, att_ref, WzT_ref, bzT_ref,
               WlzT_ref, blzT_ref, WhT_ref, bhT_ref, WlhT_ref, blhT_ref,
               WlinT_ref, blin_ref, out_ref):
    a = att_ref[...]
    e = jnp.exp(a - jnp.max(a))
    pr = e / jnp.sum(e)
    uz = jnp.dot(WlzT_ref[...], WzT_ref[...])
    cz = jnp.dot(WlzT_ref[...], bzT_ref[...]) + blzT_ref[...]
    uh = jnp.dot(WlhT_ref[...], WhT_ref[...])
    ch = jnp.dot(WlhT_ref[...], bhT_ref[...]) + blhT_ref[...]
    t = t0_ref[...] + t1_ref[...] + yt_ref[...]
    s_all = dinv_ref[...] * t
    h = jnp.zeros((OUT, BL), jnp.float32)
    for p in range(P):
        sp = s_all[p:p + 1, :]
        z = jax.nn.sigmoid(uz * sp + cz)
        ht = jnp.tanh(uh * sp + ch)
        h = h + pr[0:1, p:p + 1] * (1.0 - z) * ht
    out_ref[...] = jnp.dot(WlinT_ref[...], jax.nn.relu(h)) + blin_ref[...]


def _gate_call(t0, t1, y_t, dinv, att2, WzT, bzT, WlzT, blzT, WhT, bhT,
               WlhT, blhT, WlinT, blin2):
    node = lambda r: pl.BlockSpec((r, BL), lambda i: (0, i))
    full = lambda shp: pl.BlockSpec(shp, lambda i: (0, 0))
    return pl.pallas_call(
        _gate_body,
        grid=(NB,),
        in_specs=[node(P), node(P), node(P), node(1),
                  full((1, P)), full((OUT, 1)), full((OUT, 1)),
                  full((OUT, OUT)), full((OUT, 1)), full((OUT, 1)),
                  full((OUT, 1)), full((OUT, OUT)), full((OUT, 1)),
                  full((1, OUT)), full((1, 1))],
        out_specs=node(1),
        out_shape=jax.ShapeDtypeStruct((1, NP), jnp.float32),
    )(t0, t1, y_t, dinv, att2, WzT, bzT, WlzT, blzT, WhT, bhT, WlhT, blhT,
      WlinT, blin2)


def kernel(x, edge_index, edge_weight, att, Wz, bz, Wr, br, Wh, bh,
           Wlz, blz, Wlr, blr, Wlh, blh, Wlin, blin):
    row = edge_index[0]
    col = edge_index[1]
    x_t = jnp.pad(x.T, ((0, 0), (0, NP - N)))

    degp = _deg_kernel(col, edge_weight)
    dinv, y_t = _norm_call(degp.reshape(2, NP), x_t)
    tp = _msg_kernel(row, col, edge_weight, y_t.reshape(4 * NP))
    out_t = _gate_call(
        tp[:4 * NP].reshape(P, NP), tp[4 * NP:].reshape(P, NP), y_t, dinv,
        att.reshape(1, P),
        Wz.reshape(OUT, 1), bz.reshape(OUT, 1),
        Wlz[:OUT].T, blz.reshape(OUT, 1),
        Wh.reshape(OUT, 1), bh.reshape(OUT, 1),
        Wlh[:OUT].T, blh.reshape(OUT, 1),
        Wlin.reshape(1, OUT), blin.reshape(1, 1))
    return out_t.reshape(NP, 1)[:N]
